# baseline (speedup 1.0000x reference)
import functools

import jax
import jax.numpy as jnp
from jax import lax
from jax.experimental import pallas as pl
from jax.experimental.pallas import tpu as pltpu
from jax.experimental.pallas import tpu_sc as plsc

_BI = 2048
_BJ = 2048
_BK = 512
_BROW = 512
_BDEG = 512

_NSUB = 16
_LANES = 128
_ZCHUNK = 65536


def _scatter_body(n_slots, n_zero, stripe,
                  i1a_ref, i1b_ref, i2a_ref, i2b_ref, zsrc_ref, osrc_ref,
                  m1a_ref, m1b_ref, m2a_ref, m2b_ref,
                  idx_va, idx_vb, ones_v, zero_v, sem):
    c = lax.axis_index("c")
    s = lax.axis_index("s")
    pltpu.sync_copy(zsrc_ref, zero_v)
    pltpu.sync_copy(osrc_ref, ones_v)
    base = s * stripe

    def zero_mat(m_ref):
        def zb(w, carry):
            for u in range(4):
                pltpu.async_copy(
                    zero_v,
                    m_ref.at[pl.ds(base + (w * 4 + u) * _ZCHUNK, _ZCHUNK)],
                    sem)
            for u in range(4):
                pltpu.make_async_copy(
                    zero_v,
                    m_ref.at[pl.ds(base + (w * 4 + u) * _ZCHUNK, _ZCHUNK)],
                    sem).wait()
            return carry
        lax.fori_loop(0, n_zero // 4, zb, 0)

    def fire(idx_v, m_ref):
        def sb(j, carry):
            pltpu.async_copy(ones_v, m_ref.at[idx_v.at[j]], sem)
            return carry
        lax.fori_loop(0, n_slots, sb, 0)

    def run_pair(ia_ref, ma_ref, ib_ref, mb_ref):
        pltpu.sync_copy(ia_ref.at[s], idx_va)
        pltpu.sync_copy(ib_ref.at[s], idx_vb)
        zero_mat(ma_ref)
        plsc.subcore_barrier()
        fire(idx_va, ma_ref)
        zero_mat(mb_ref)
        plsc.subcore_barrier()
        fire(idx_vb, mb_ref)

        def drain(j, carry):
            pltpu.make_async_copy(ones_v, ma_ref.at[idx_va.at[0]], sem).wait()
            return carry
        lax.fori_loop(0, 2 * n_slots, drain, 0)

    @pl.when(c == 0)
    def _run0():
        run_pair(i1a_ref, m1a_ref, i1b_ref, m1b_ref)

    @pl.when(c == 1)
    def _run1():
        run_pair(i2a_ref, m2a_ref, i2b_ref, m2b_ref)


def _densify_sc(e1a, e1b, e2a, e2b, n_pad):
    m_elems = n_pad * n_pad
    stripe = m_elems // _NSUB
    n_zero = stripe // _ZCHUNK
    e_count = e1a.shape[1]
    n_slots = -(-e_count // (_NSUB * _LANES))
    total = _NSUB * n_slots * _LANES

    def flat_idx(e):
        f = e[0] * n_pad + e[1]
        pad = jnp.broadcast_to(f[0], (total - e_count,))
        return jnp.concatenate([f, pad]).reshape(_NSUB, n_slots, _LANES)

    idxs = [flat_idx(e) for e in (e1a, e1b, e2a, e2b)]
    zsrc = jnp.zeros((_ZCHUNK,), jnp.float32)
    osrc = jnp.ones((_LANES,), jnp.float32)

    mat = jax.ShapeDtypeStruct((m_elems,), jnp.float32)
    kfn = pl.kernel(
        functools.partial(_scatter_body, n_slots, n_zero, stripe),
        out_type=[mat, mat, mat, mat],
        mesh=plsc.VectorSubcoreMesh(core_axis_name="c", subcore_axis_name="s"),
        scratch_types=[
            pltpu.VMEM((n_slots, _LANES), jnp.int32),
            pltpu.VMEM((n_slots, _LANES), jnp.int32),
            pltpu.VMEM((_LANES,), jnp.float32),
            pltpu.VMEM((_ZCHUNK,), jnp.float32),
            pltpu.SemaphoreType.DMA,
        ],
    )
    m1a, m1b, m2a, m2b = kfn(idxs[0], idxs[1], idxs[2], idxs[3], zsrc, osrc)
    shape2 = (n_pad, n_pad)
    return (m1a.reshape(shape2), m1b.reshape(shape2),
            m2a.reshape(shape2), m2b.reshape(shape2))


def _spspmm_body(nk, ma_ref, mb_ref, adj_ref, deg_ref, acc_ref):
    i = pl.program_id(1)
    k = pl.program_id(2)

    @pl.when(k == 0)
    def _init():
        acc_ref[...] = jnp.zeros_like(acc_ref)

    acc_ref[...] += jnp.dot(ma_ref[...].astype(jnp.bfloat16),
                            mb_ref[...].astype(jnp.bfloat16),
                            preferred_element_type=jnp.float32)

    @pl.when(k == nk - 1)
    def _finish():
        adjf = (acc_ref[...] > 0.0).astype(jnp.float32)
        adj_ref[...] = adjf.astype(jnp.bfloat16)
        cs = jnp.sum(adjf, axis=0, keepdims=True)

        @pl.when(i == 0)
        def _set():
            deg_ref[...] = cs

        @pl.when(i != 0)
        def _acc():
            deg_ref[...] += cs


def _spspmm_deg(ma, mb, n_pad):
    nj, ni, nk = n_pad // _BJ, n_pad // _BI, n_pad // _BK
    return pl.pallas_call(
        functools.partial(_spspmm_body, nk),
        grid=(nj, ni, nk),
        in_specs=[
            pl.BlockSpec((_BI, _BK), lambda j, i, k: (i, k)),
            pl.BlockSpec((_BK, _BJ), lambda j, i, k: (k, j)),
        ],
        out_specs=[
            pl.BlockSpec((_BI, _BJ), lambda j, i, k: (i, j)),
            pl.BlockSpec((1, _BJ), lambda j, i, k: (0, j)),
        ],
        out_shape=[
            jax.ShapeDtypeStruct((n_pad, n_pad), jnp.bfloat16),
            jax.ShapeDtypeStruct((1, n_pad), jnp.float32),
        ],
        scratch_shapes=[pltpu.VMEM((_BI, _BJ), jnp.float32)],
        compiler_params=pltpu.CompilerParams(
            dimension_semantics=("parallel", "arbitrary", "arbitrary")),
    )(ma, mb)


def _y_body(x_ref, w_ref, b_ref, d1_ref, d2_ref,
            y1_ref, y2_ref, v1_ref, v2_ref):
    h = jnp.maximum(
        jnp.dot(x_ref[...], w_ref[...], preferred_element_type=jnp.float32)
        + b_ref[...], 0.0)
    v1 = jax.lax.rsqrt(d1_ref[...] + 1.0)
    v2 = jax.lax.rsqrt(d2_ref[...] + 1.0)
    y1_ref[...] = h * v1
    y2_ref[...] = h * v2
    v1_ref[...] = v1
    v2_ref[...] = v2


def _features(x_pad, w, b_row, deg1c, deg2c):
    n, hdim = x_pad.shape[0], w.shape[1]
    return pl.pallas_call(
        _y_body,
        out_shape=[
            jax.ShapeDtypeStruct((n, hdim), jnp.float32),
            jax.ShapeDtypeStruct((n, hdim), jnp.float32),
            jax.ShapeDtypeStruct((n, 1), jnp.float32),
            jax.ShapeDtypeStruct((n, 1), jnp.float32),
        ],
    )(x_pad, w, b_row, deg1c, deg2c)


def _attend_body(bi, adj1_ref, adj2_ref, y1_ref, y2_ref, v1_ref, v2_ref,
                 wa_ref, wo_ref, bo_ref, out_ref):
    i = pl.program_id(0)
    y1 = y1_ref[...]
    y2 = y2_ref[...]
    a1 = jnp.dot(adj1_ref[...].astype(jnp.float32), y1,
                 preferred_element_type=jnp.float32)
    a2 = jnp.dot(adj2_ref[...].astype(jnp.float32), y2,
                 preferred_element_type=jnp.float32)
    rows = pl.ds(i * bi, bi)
    agg1 = v1_ref[...] * (a1 + y1_ref[rows, :])
    agg2 = v2_ref[...] * (a2 + y2_ref[rows, :])
    wa = wa_ref[...]
    s1 = jnp.dot(jnp.tanh(agg1), wa, preferred_element_type=jnp.float32)
    s2 = jnp.dot(jnp.tanh(agg2), wa, preferred_element_type=jnp.float32)
    mx = jnp.maximum(s1, s2)
    e1 = jnp.exp(s1 - mx)
    e2 = jnp.exp(s2 - mx)
    z = e1 + e2
    fin = (e1 / z) * agg1 + (e2 / z) * agg2
    out_ref[...] = jnp.dot(fin, wo_ref[...],
                           preferred_element_type=jnp.float32) + bo_ref[...]


def _aggregate_attend(adj1, adj2, y1, y2, v1, v2, watt, wout, bout_row, n_pad):
    hdim = y1.shape[1]
    dout = wout.shape[1]
    ni = n_pad // _BROW
    return pl.pallas_call(
        functools.partial(_attend_body, _BROW),
        grid=(ni,),
        in_specs=[
            pl.BlockSpec((_BROW, n_pad), lambda i: (i, 0)),
            pl.BlockSpec((_BROW, n_pad), lambda i: (i, 0)),
            pl.BlockSpec((n_pad, hdim), lambda i: (0, 0)),
            pl.BlockSpec((n_pad, hdim), lambda i: (0, 0)),
            pl.BlockSpec((_BROW, 1), lambda i: (i, 0)),
            pl.BlockSpec((_BROW, 1), lambda i: (i, 0)),
            pl.BlockSpec(watt.shape, lambda i: (0, 0)),
            pl.BlockSpec(wout.shape, lambda i: (0, 0)),
            pl.BlockSpec((1, dout), lambda i: (0, 0)),
        ],
        out_specs=pl.BlockSpec((_BROW, dout), lambda i: (i, 0)),
        out_shape=jax.ShapeDtypeStruct((n_pad, dout), jnp.float32),
        compiler_params=pltpu.CompilerParams(
            dimension_semantics=("arbitrary",)),
    )(adj1, adj2, y1, y2, v1, v2, watt, wout, bout_row)


def kernel(x_customer, x_product, edge_index_buys, edge_index_bought_by,
           edge_index_views, edge_index_viewed_by, W_customer, b_customer,
           W_product, b_product, W_att, W_out, b_out):
    n_c = x_customer.shape[0]
    n_pad = ((n_c + _BI - 1) // _BI) * _BI

    ma1, mb1, ma2, mb2 = _densify_sc(
        edge_index_buys, edge_index_bought_by,
        edge_index_views, edge_index_viewed_by, n_pad)
    adj1, deg1 = _spspmm_deg(ma1, mb1, n_pad)
    adj2, deg2 = _spspmm_deg(ma2, mb2, n_pad)

    x_pad = jnp.concatenate(
        [x_customer, jnp.zeros((n_pad - n_c, x_customer.shape[1]),
                               x_customer.dtype)], axis=0)
    y1, y2, v1, v2 = _features(
        x_pad, W_customer, b_customer.reshape(1, -1),
        deg1.reshape(n_pad, 1), deg2.reshape(n_pad, 1))

    out = _aggregate_attend(adj1, adj2, y1, y2, v1, v2,
                            W_att, W_out, b_out.reshape(1, -1), n_pad)
    return out[:n_c]


# speedup vs baseline: 1.2143x; 1.2143x over previous
import functools

import jax
import jax.numpy as jnp
from jax import lax
from jax.experimental import pallas as pl
from jax.experimental.pallas import tpu as pltpu
from jax.experimental.pallas import tpu_sc as plsc

_BI = 2048
_BJ = 2048
_BK = 512
_BROW = 512
_BDEG = 512

_NSUB = 16
_LANES = 128
_ZCHUNK = 65536


def _scatter_body(n_slots, n_zero, stripe,
                  i1a_ref, i1b_ref, i2a_ref, i2b_ref, zsrc_ref, osrc_ref,
                  m1a_ref, m1b_ref, m2a_ref, m2b_ref,
                  idx_va, idx_vb, ones_v, zero_v, sem):
    c = lax.axis_index("c")
    s = lax.axis_index("s")
    pltpu.sync_copy(zsrc_ref, zero_v)
    pltpu.sync_copy(osrc_ref, ones_v)
    base = s * stripe

    def zero_mat(m_ref):
        def zb(t, carry):
            pltpu.sync_copy(zero_v,
                            m_ref.at[pl.ds(base + t * _ZCHUNK, _ZCHUNK)])
            return carry
        lax.fori_loop(0, n_zero, zb, 0)

    def scat(idx_v, i_ref, m_ref):
        pltpu.sync_copy(i_ref.at[s], idx_v)

        def sb(j, carry):
            pltpu.async_copy(ones_v, m_ref.at[idx_v.at[j]], sem).wait()
            return carry
        lax.fori_loop(0, n_slots, sb, 0)

    def run_pair(ia_ref, ma_ref, ib_ref, mb_ref):
        zero_mat(ma_ref)
        zero_mat(mb_ref)
        plsc.subcore_barrier()
        scat(idx_va, ia_ref, ma_ref)
        scat(idx_vb, ib_ref, mb_ref)

    @pl.when(c == 0)
    def _run0():
        run_pair(i1a_ref, m1a_ref, i1b_ref, m1b_ref)

    @pl.when(c == 1)
    def _run1():
        run_pair(i2a_ref, m2a_ref, i2b_ref, m2b_ref)


def _densify_sc(e1a, e1b, e2a, e2b, n_pad):
    m_elems = n_pad * n_pad
    stripe = m_elems // _NSUB
    n_zero = stripe // _ZCHUNK
    e_count = e1a.shape[1]
    n_slots = -(-e_count // (_NSUB * _LANES))
    total = _NSUB * n_slots * _LANES

    def flat_idx(e):
        f = e[0] * n_pad + e[1]
        pad = jnp.broadcast_to(f[0], (total - e_count,))
        return jnp.concatenate([f, pad]).reshape(_NSUB, n_slots, _LANES)

    idxs = [flat_idx(e) for e in (e1a, e1b, e2a, e2b)]
    zsrc = jnp.zeros((_ZCHUNK,), jnp.float32)
    osrc = jnp.ones((_LANES,), jnp.float32)

    mat = jax.ShapeDtypeStruct((m_elems,), jnp.float32)
    kfn = pl.kernel(
        functools.partial(_scatter_body, n_slots, n_zero, stripe),
        out_type=[mat, mat, mat, mat],
        mesh=plsc.VectorSubcoreMesh(core_axis_name="c", subcore_axis_name="s"),
        scratch_types=[
            pltpu.VMEM((n_slots, _LANES), jnp.int32),
            pltpu.VMEM((n_slots, _LANES), jnp.int32),
            pltpu.VMEM((_LANES,), jnp.float32),
            pltpu.VMEM((_ZCHUNK,), jnp.float32),
            pltpu.SemaphoreType.DMA,
        ],
    )
    m1a, m1b, m2a, m2b = kfn(idxs[0], idxs[1], idxs[2], idxs[3], zsrc, osrc)
    shape2 = (n_pad, n_pad)
    return (m1a.reshape(shape2), m1b.reshape(shape2),
            m2a.reshape(shape2), m2b.reshape(shape2))


def _spspmm_body(nk, ma_ref, mb_ref, adj_ref, deg_ref, acc_ref):
    i = pl.program_id(1)
    k = pl.program_id(2)

    @pl.when(k == 0)
    def _init():
        acc_ref[...] = jnp.zeros_like(acc_ref)

    acc_ref[...] += jnp.dot(ma_ref[...].astype(jnp.float8_e4m3fn),
                            mb_ref[...].astype(jnp.float8_e4m3fn),
                            preferred_element_type=jnp.float32)

    @pl.when(k == nk - 1)
    def _finish():
        adjf = (acc_ref[...] > 0.0).astype(jnp.float32)
        adj_ref[...] = adjf.astype(jnp.bfloat16)
        cs = jnp.sum(adjf, axis=0, keepdims=True)

        @pl.when(i == 0)
        def _set():
            deg_ref[...] = cs

        @pl.when(i != 0)
        def _acc():
            deg_ref[...] += cs


def _spspmm_deg(ma, mb, n_pad):
    nj, ni, nk = n_pad // _BJ, n_pad // _BI, n_pad // _BK
    return pl.pallas_call(
        functools.partial(_spspmm_body, nk),
        grid=(nj, ni, nk),
        in_specs=[
            pl.BlockSpec((_BI, _BK), lambda j, i, k: (i, k)),
            pl.BlockSpec((_BK, _BJ), lambda j, i, k: (k, j)),
        ],
        out_specs=[
            pl.BlockSpec((_BI, _BJ), lambda j, i, k: (i, j)),
            pl.BlockSpec((1, _BJ), lambda j, i, k: (0, j)),
        ],
        out_shape=[
            jax.ShapeDtypeStruct((n_pad, n_pad), jnp.bfloat16),
            jax.ShapeDtypeStruct((1, n_pad), jnp.float32),
        ],
        scratch_shapes=[pltpu.VMEM((_BI, _BJ), jnp.float32)],
        compiler_params=pltpu.CompilerParams(
            dimension_semantics=("parallel", "arbitrary", "arbitrary")),
    )(ma, mb)


def _y_body(x_ref, w_ref, b_ref, d1_ref, d2_ref,
            y1_ref, y2_ref, v1_ref, v2_ref):
    h = jnp.maximum(
        jnp.dot(x_ref[...], w_ref[...], preferred_element_type=jnp.float32)
        + b_ref[...], 0.0)
    v1 = jax.lax.rsqrt(d1_ref[...] + 1.0)
    v2 = jax.lax.rsqrt(d2_ref[...] + 1.0)
    y1_ref[...] = h * v1
    y2_ref[...] = h * v2
    v1_ref[...] = v1
    v2_ref[...] = v2


def _features(x_pad, w, b_row, deg1c, deg2c):
    n, hdim = x_pad.shape[0], w.shape[1]
    return pl.pallas_call(
        _y_body,
        out_shape=[
            jax.ShapeDtypeStruct((n, hdim), jnp.float32),
            jax.ShapeDtypeStruct((n, hdim), jnp.float32),
            jax.ShapeDtypeStruct((n, 1), jnp.float32),
            jax.ShapeDtypeStruct((n, 1), jnp.float32),
        ],
    )(x_pad, w, b_row, deg1c, deg2c)


def _attend_body(bi, adj1_ref, adj2_ref, y1_ref, y2_ref, v1_ref, v2_ref,
                 wa_ref, wo_ref, bo_ref, out_ref):
    i = pl.program_id(0)
    y1 = y1_ref[...]
    y2 = y2_ref[...]
    a1 = jnp.dot(adj1_ref[...].astype(jnp.float32), y1,
                 preferred_element_type=jnp.float32)
    a2 = jnp.dot(adj2_ref[...].astype(jnp.float32), y2,
                 preferred_element_type=jnp.float32)
    rows = pl.ds(i * bi, bi)
    agg1 = v1_ref[...] * (a1 + y1_ref[rows, :])
    agg2 = v2_ref[...] * (a2 + y2_ref[rows, :])
    wa = wa_ref[...]
    s1 = jnp.dot(jnp.tanh(agg1), wa, preferred_element_type=jnp.float32)
    s2 = jnp.dot(jnp.tanh(agg2), wa, preferred_element_type=jnp.float32)
    mx = jnp.maximum(s1, s2)
    e1 = jnp.exp(s1 - mx)
    e2 = jnp.exp(s2 - mx)
    z = e1 + e2
    fin = (e1 / z) * agg1 + (e2 / z) * agg2
    out_ref[...] = jnp.dot(fin, wo_ref[...],
                           preferred_element_type=jnp.float32) + bo_ref[...]


def _aggregate_attend(adj1, adj2, y1, y2, v1, v2, watt, wout, bout_row, n_pad):
    hdim = y1.shape[1]
    dout = wout.shape[1]
    ni = n_pad // _BROW
    return pl.pallas_call(
        functools.partial(_attend_body, _BROW),
        grid=(ni,),
        in_specs=[
            pl.BlockSpec((_BROW, n_pad), lambda i: (i, 0)),
            pl.BlockSpec((_BROW, n_pad), lambda i: (i, 0)),
            pl.BlockSpec((n_pad, hdim), lambda i: (0, 0)),
            pl.BlockSpec((n_pad, hdim), lambda i: (0, 0)),
            pl.BlockSpec((_BROW, 1), lambda i: (i, 0)),
            pl.BlockSpec((_BROW, 1), lambda i: (i, 0)),
            pl.BlockSpec(watt.shape, lambda i: (0, 0)),
            pl.BlockSpec(wout.shape, lambda i: (0, 0)),
            pl.BlockSpec((1, dout), lambda i: (0, 0)),
        ],
        out_specs=pl.BlockSpec((_BROW, dout), lambda i: (i, 0)),
        out_shape=jax.ShapeDtypeStruct((n_pad, dout), jnp.float32),
        compiler_params=pltpu.CompilerParams(
            dimension_semantics=("arbitrary",)),
    )(adj1, adj2, y1, y2, v1, v2, watt, wout, bout_row)


def kernel(x_customer, x_product, edge_index_buys, edge_index_bought_by,
           edge_index_views, edge_index_viewed_by, W_customer, b_customer,
           W_product, b_product, W_att, W_out, b_out):
    n_c = x_customer.shape[0]
    n_pad = ((n_c + _BI - 1) // _BI) * _BI

    ma1, mb1, ma2, mb2 = _densify_sc(
        edge_index_buys, edge_index_bought_by,
        edge_index_views, edge_index_viewed_by, n_pad)
    adj1, deg1 = _spspmm_deg(ma1, mb1, n_pad)
    adj2, deg2 = _spspmm_deg(ma2, mb2, n_pad)

    x_pad = jnp.concatenate(
        [x_customer, jnp.zeros((n_pad - n_c, x_customer.shape[1]),
                               x_customer.dtype)], axis=0)
    y1, y2, v1, v2 = _features(
        x_pad, W_customer, b_customer.reshape(1, -1),
        deg1.reshape(n_pad, 1), deg2.reshape(n_pad, 1))

    out = _aggregate_attend(adj1, adj2, y1, y2, v1, v2,
                            W_att, W_out, b_out.reshape(1, -1), n_pad)
    return out[:n_c]


# speedup vs baseline: 1.2628x; 1.0400x over previous
import functools

import jax
import jax.numpy as jnp
from jax import lax
from jax.experimental import pallas as pl
from jax.experimental.pallas import tpu as pltpu
from jax.experimental.pallas import tpu_sc as plsc

_BI = 2048
_BJ = 2048
_BK = 512
_BROW = 512
_BDEG = 512

_NSUB = 16
_LANES = 128
_ZCHUNK = 65536


def _scatter_body(n_slots, n_zero, stripe,
                  ia_ref, ib_ref, zsrc_ref, osrc_ref,
                  ma_ref, mb_ref, idx_v, ones_v, zero_v, sem):
    c = lax.axis_index("c")
    s = lax.axis_index("s")
    pltpu.sync_copy(zsrc_ref, zero_v)
    pltpu.sync_copy(osrc_ref, ones_v)
    base = s * stripe

    def zero_mat(m_ref):
        def zb(t, carry):
            pltpu.sync_copy(zero_v,
                            m_ref.at[pl.ds(base + t * _ZCHUNK, _ZCHUNK)])
            return carry
        lax.fori_loop(0, n_zero, zb, 0)

    def scat(i_ref, m_ref):
        pltpu.sync_copy(i_ref.at[s], idx_v)

        def sb(j, carry):
            pltpu.async_copy(ones_v, m_ref.at[idx_v.at[j]], sem).wait()
            return carry
        lax.fori_loop(0, n_slots, sb, 0)

    @pl.when(c == 0)
    def _z0():
        zero_mat(ma_ref)

    @pl.when(c == 1)
    def _z1():
        zero_mat(mb_ref)

    plsc.subcore_barrier()

    @pl.when(c == 0)
    def _s0():
        scat(ia_ref, ma_ref)

    @pl.when(c == 1)
    def _s1():
        scat(ib_ref, mb_ref)


def _densify_pair(ea, eb, n_pad):
    m_elems = n_pad * n_pad
    stripe = m_elems // _NSUB
    n_zero = stripe // _ZCHUNK
    e_count = ea.shape[1]
    n_slots = -(-e_count // (_NSUB * _LANES))
    total = _NSUB * n_slots * _LANES

    def flat_idx(e):
        f = e[0] * n_pad + e[1]
        pad = jnp.broadcast_to(f[0], (total - e_count,))
        return jnp.concatenate([f, pad]).reshape(_NSUB, n_slots, _LANES)

    zsrc = jnp.zeros((_ZCHUNK,), jnp.float32)
    osrc = jnp.ones((_LANES,), jnp.float32)

    mat = jax.ShapeDtypeStruct((m_elems,), jnp.float32)
    kfn = pl.kernel(
        functools.partial(_scatter_body, n_slots, n_zero, stripe),
        out_type=[mat, mat],
        mesh=plsc.VectorSubcoreMesh(core_axis_name="c", subcore_axis_name="s"),
        scratch_types=[
            pltpu.VMEM((n_slots, _LANES), jnp.int32),
            pltpu.VMEM((_LANES,), jnp.float32),
            pltpu.VMEM((_ZCHUNK,), jnp.float32),
            pltpu.SemaphoreType.DMA,
        ],
    )
    ma, mb = kfn(flat_idx(ea), flat_idx(eb), zsrc, osrc)
    shape2 = (n_pad, n_pad)
    return ma.reshape(shape2), mb.reshape(shape2)


def _spspmm_body(nk, ma_ref, mb_ref, adj_ref, deg_ref, acc_ref):
    i = pl.program_id(1)
    k = pl.program_id(2)

    @pl.when(k == 0)
    def _init():
        acc_ref[...] = jnp.zeros_like(acc_ref)

    acc_ref[...] += jnp.dot(ma_ref[...].astype(jnp.float8_e4m3fn),
                            mb_ref[...].astype(jnp.float8_e4m3fn),
                            preferred_element_type=jnp.float32)

    @pl.when(k == nk - 1)
    def _finish():
        adjf = (acc_ref[...] > 0.0).astype(jnp.float32)
        adj_ref[...] = adjf.astype(jnp.bfloat16)
        cs = jnp.sum(adjf, axis=0, keepdims=True)

        @pl.when(i == 0)
        def _set():
            deg_ref[...] = cs

        @pl.when(i != 0)
        def _acc():
            deg_ref[...] += cs


def _spspmm_deg(ma, mb, n_pad):
    nj, ni, nk = n_pad // _BJ, n_pad // _BI, n_pad // _BK
    return pl.pallas_call(
        functools.partial(_spspmm_body, nk),
        grid=(nj, ni, nk),
        in_specs=[
            pl.BlockSpec((_BI, _BK), lambda j, i, k: (i, k)),
            pl.BlockSpec((_BK, _BJ), lambda j, i, k: (k, j)),
        ],
        out_specs=[
            pl.BlockSpec((_BI, _BJ), lambda j, i, k: (i, j)),
            pl.BlockSpec((1, _BJ), lambda j, i, k: (0, j)),
        ],
        out_shape=[
            jax.ShapeDtypeStruct((n_pad, n_pad), jnp.bfloat16),
            jax.ShapeDtypeStruct((1, n_pad), jnp.float32),
        ],
        scratch_shapes=[pltpu.VMEM((_BI, _BJ), jnp.float32)],
        compiler_params=pltpu.CompilerParams(
            dimension_semantics=("parallel", "arbitrary", "arbitrary")),
    )(ma, mb)


def _y_body(x_ref, w_ref, b_ref, d1_ref, d2_ref,
            y1_ref, y2_ref, v1_ref, v2_ref):
    h = jnp.maximum(
        jnp.dot(x_ref[...], w_ref[...], preferred_element_type=jnp.float32)
        + b_ref[...], 0.0)
    v1 = jax.lax.rsqrt(d1_ref[...] + 1.0)
    v2 = jax.lax.rsqrt(d2_ref[...] + 1.0)
    y1_ref[...] = h * v1
    y2_ref[...] = h * v2
    v1_ref[...] = v1
    v2_ref[...] = v2


def _features(x_pad, w, b_row, deg1c, deg2c):
    n, hdim = x_pad.shape[0], w.shape[1]
    return pl.pallas_call(
        _y_body,
        out_shape=[
            jax.ShapeDtypeStruct((n, hdim), jnp.float32),
            jax.ShapeDtypeStruct((n, hdim), jnp.float32),
            jax.ShapeDtypeStruct((n, 1), jnp.float32),
            jax.ShapeDtypeStruct((n, 1), jnp.float32),
        ],
    )(x_pad, w, b_row, deg1c, deg2c)


def _attend_body(bi, adj1_ref, adj2_ref, y1_ref, y2_ref, v1_ref, v2_ref,
                 wa_ref, wo_ref, bo_ref, out_ref):
    i = pl.program_id(0)
    y1 = y1_ref[...]
    y2 = y2_ref[...]
    a1 = jnp.dot(adj1_ref[...].astype(jnp.float32), y1,
                 preferred_element_type=jnp.float32)
    a2 = jnp.dot(adj2_ref[...].astype(jnp.float32), y2,
                 preferred_element_type=jnp.float32)
    rows = pl.ds(i * bi, bi)
    agg1 = v1_ref[...] * (a1 + y1_ref[rows, :])
    agg2 = v2_ref[...] * (a2 + y2_ref[rows, :])
    wa = wa_ref[...]
    s1 = jnp.dot(jnp.tanh(agg1), wa, preferred_element_type=jnp.float32)
    s2 = jnp.dot(jnp.tanh(agg2), wa, preferred_element_type=jnp.float32)
    mx = jnp.maximum(s1, s2)
    e1 = jnp.exp(s1 - mx)
    e2 = jnp.exp(s2 - mx)
    z = e1 + e2
    fin = (e1 / z) * agg1 + (e2 / z) * agg2
    out_ref[...] = jnp.dot(fin, wo_ref[...],
                           preferred_element_type=jnp.float32) + bo_ref[...]


def _aggregate_attend(adj1, adj2, y1, y2, v1, v2, watt, wout, bout_row, n_pad):
    hdim = y1.shape[1]
    dout = wout.shape[1]
    ni = n_pad // _BROW
    return pl.pallas_call(
        functools.partial(_attend_body, _BROW),
        grid=(ni,),
        in_specs=[
            pl.BlockSpec((_BROW, n_pad), lambda i: (i, 0)),
            pl.BlockSpec((_BROW, n_pad), lambda i: (i, 0)),
            pl.BlockSpec((n_pad, hdim), lambda i: (0, 0)),
            pl.BlockSpec((n_pad, hdim), lambda i: (0, 0)),
            pl.BlockSpec((_BROW, 1), lambda i: (i, 0)),
            pl.BlockSpec((_BROW, 1), lambda i: (i, 0)),
            pl.BlockSpec(watt.shape, lambda i: (0, 0)),
            pl.BlockSpec(wout.shape, lambda i: (0, 0)),
            pl.BlockSpec((1, dout), lambda i: (0, 0)),
        ],
        out_specs=pl.BlockSpec((_BROW, dout), lambda i: (i, 0)),
        out_shape=jax.ShapeDtypeStruct((n_pad, dout), jnp.float32),
        compiler_params=pltpu.CompilerParams(
            dimension_semantics=("arbitrary",)),
    )(adj1, adj2, y1, y2, v1, v2, watt, wout, bout_row)


def kernel(x_customer, x_product, edge_index_buys, edge_index_bought_by,
           edge_index_views, edge_index_viewed_by, W_customer, b_customer,
           W_product, b_product, W_att, W_out, b_out):
    n_c = x_customer.shape[0]
    n_pad = ((n_c + _BI - 1) // _BI) * _BI

    ma1, mb1 = _densify_pair(edge_index_buys, edge_index_bought_by, n_pad)
    ma2, mb2 = _densify_pair(edge_index_views, edge_index_viewed_by, n_pad)
    adj1, deg1 = _spspmm_deg(ma1, mb1, n_pad)
    adj2, deg2 = _spspmm_deg(ma2, mb2, n_pad)

    x_pad = jnp.concatenate(
        [x_customer, jnp.zeros((n_pad - n_c, x_customer.shape[1]),
                               x_customer.dtype)], axis=0)
    y1, y2, v1, v2 = _features(
        x_pad, W_customer, b_customer.reshape(1, -1),
        deg1.reshape(n_pad, 1), deg2.reshape(n_pad, 1))

    out = _aggregate_attend(adj1, adj2, y1, y2, v1, v2,
                            W_att, W_out, b_out.reshape(1, -1), n_pad)
    return out[:n_c]
